# baseline (speedup 1.0000x reference)
import functools

import jax
import jax.numpy as jnp
from jax import lax
from jax.experimental import pallas as pl
from jax.experimental.pallas import tpu as pltpu
from jax.experimental.pallas import tpu_sc as plsc

_NW = 32
_NB = 8
_LA = 4
_TC = 800


def _tr_call(V, D):
    n_ch = (V + _TC - 1) // _TC
    n_steps = (n_ch + _NW - 1) // _NW
    assert n_steps % 2 == 0 and n_steps >= 4
    mesh = plsc.VectorSubcoreMesh(core_axis_name="c", subcore_axis_name="s")

    @functools.partial(
        pl.kernel,
        mesh=mesh,
        out_type=jax.ShapeDtypeStruct((V, D), jnp.float32),
        scratch_types=[
            pltpu.VMEM((2, D, _TC), jnp.float32),
            pltpu.VMEM((2, _TC, D), jnp.float32),
            [pltpu.SemaphoreType.DMA] * 2,
            [pltpu.SemaphoreType.DMA] * 2,
        ],
        compiler_params=pltpu.CompilerParams(
            use_tc_tiling_on_sc=False, needs_layout_passes=False),
    )
    def tr(et_hbm, out_hbm, in2, out2, isems, osems):
        wid = lax.axis_index("s") * 2 + lax.axis_index("c")

        def r0(i):
            return jnp.minimum(wid + _NW * i, n_ch - 1) * _TC

        def i_fire(i, p):
            pltpu.async_copy(
                et_hbm.at[:, pl.ds(r0(i), _TC)], in2.at[p], isems[p])

        def i_wait(i, p):
            pltpu.make_async_copy(
                et_hbm.at[:, pl.ds(r0(i), _TC)], in2.at[p], isems[p]).wait()

        def o_fire(i, p):
            pltpu.async_copy(
                out2.at[p], out_hbm.at[pl.ds(r0(i), _TC)], osems[p])

        def o_wait(i, p):
            pltpu.make_async_copy(
                out2.at[p], out_hbm.at[pl.ds(r0(i), _TC)], osems[p]).wait()

        iota = lax.iota(jnp.int32, 16)
        dvecs = [jnp.full((16,), d, jnp.int32) for d in range(D)]

        def compute(p):
            def kbody(k, carry):
                row = iota + k * 16
                for d in range(D):
                    x = in2[p, d, pl.ds(k * 16, 16)]
                    plsc.store_scatter(out2.at[p], [row, dvecs[d]], x)
                return carry
            lax.fori_loop(0, _TC // 16, kbody, 0)

        def step(i, p, first, last):
            if not last:
                i_fire(i + 1, 1 - p)
            i_wait(i, p)
            if not first:
                o_wait(i - 2, p)
            compute(p)
            o_fire(i, p)

        i_fire(0, 0)
        step(0, 0, first=True, last=False)
        step(1, 1, first=True, last=False)

        def pair(g, carry):
            step(g * 2, 0, first=False, last=False)
            step(g * 2 + 1, 1, first=False, last=False)
            return carry

        lax.fori_loop(1, n_steps // 2 - 1, pair, 0)

        step(n_steps - 2, 0, first=False, last=False)
        step(n_steps - 1, 1, first=False, last=True)
        o_wait(n_steps - 2, 0)
        o_wait(n_steps - 1, 1)

    return tr


def _emb_call(M, S, D, DP):
    m_per_w = M // _NW
    assert m_per_w % _NB == 0 and m_per_w >= 2 * _NB
    mesh = plsc.VectorSubcoreMesh(core_axis_name="c", subcore_axis_name="s")

    @functools.partial(
        pl.kernel,
        mesh=mesh,
        out_type=jax.ShapeDtypeStruct((M, S, DP), jnp.float32),
        scratch_types=[
            pltpu.VMEM((m_per_w, S), jnp.int32),
            pltpu.VMEM((_NB, S, D), jnp.float32),
            [pltpu.SemaphoreType.DMA] * _NB,
            [pltpu.SemaphoreType.DMA] * _NB,
        ],
        compiler_params=pltpu.CompilerParams(use_tc_tiling_on_sc=False),
    )
    def emb(table_hbm, idx_hbm, out_hbm, idx_v, rows_v, gsems, osems):
        wid = lax.axis_index("s") * 2 + lax.axis_index("c")
        base = wid * m_per_w
        pltpu.sync_copy(idx_hbm.at[pl.ds(base, m_per_w)], idx_v)

        def g_fire(j, b):
            pltpu.async_copy(table_hbm.at[idx_v.at[j]], rows_v.at[b], gsems[b])

        def g_wait(j, b):
            pltpu.make_async_copy(
                table_hbm.at[idx_v.at[j]], rows_v.at[b], gsems[b]).wait()

        def o_fire(j, b):
            pltpu.async_copy(
                rows_v.at[b], out_hbm.at[base + j].at[:, pl.ds(0, D)],
                osems[b])

        def o_wait(j, b):
            pltpu.make_async_copy(
                rows_v.at[b], out_hbm.at[base + j].at[:, pl.ds(0, D)],
                osems[b]).wait()

        for b in range(_LA):
            g_fire(b, b)

        def step(j, b, first, last):
            b2 = (b + _LA) % _NB
            g_wait(j, b)
            o_fire(j, b)
            if not first:
                o_wait(j - _LA, b2)
            if not last:
                g_fire(j + _LA, b2)

        for b in range(_NB):
            step(b, b, first=(b < _LA), last=False)

        def group(g, carry):
            j0 = g * _NB
            for b in range(_NB):
                step(j0 + b, b, first=False, last=False)
            return carry

        lax.fori_loop(1, m_per_w // _NB - 1, group, 0)

        j0 = m_per_w - _NB
        for b in range(_NB):
            step(j0 + b, b, first=False, last=(b >= _NB - _LA))

        for j in range(m_per_w - _LA, m_per_w):
            o_wait(j, j % _NB)

    return emb


def kernel(X, E):
    M, S = X.shape
    V, D = E.shape
    table_lin = _tr_call(V, D)(E.T)
    out_pad = _emb_call(M, S, D, 128)(table_lin, X)
    return lax.slice(out_pad, (0, 0, 0), (M, S, D))


# speedup vs baseline: 4.6832x; 4.6832x over previous
import functools

import jax
import jax.numpy as jnp
from jax import lax
from jax.experimental import pallas as pl
from jax.experimental.pallas import tpu as pltpu
from jax.experimental.pallas import tpu_sc as plsc

_NW = 32
_NB = 8
_LA = 4


def _emb_call(M, S, D, DP):
    m_per_w = M // _NW
    assert m_per_w % _NB == 0 and m_per_w >= 2 * _NB
    mesh = plsc.VectorSubcoreMesh(core_axis_name="c", subcore_axis_name="s")

    @functools.partial(
        pl.kernel,
        mesh=mesh,
        out_type=jax.ShapeDtypeStruct((M, S, DP), jnp.float32),
        scratch_types=[
            pltpu.VMEM((m_per_w, S), jnp.int32),
            pltpu.VMEM((_NB, S, D), jnp.float32),
            [pltpu.SemaphoreType.DMA] * _NB,
            [pltpu.SemaphoreType.DMA] * _NB,
        ],
        compiler_params=pltpu.CompilerParams(use_tc_tiling_on_sc=False),
    )
    def emb(table_hbm, idx_hbm, out_hbm, idx_v, rows_v, gsems, osems):
        wid = lax.axis_index("s") * 2 + lax.axis_index("c")
        base = wid * m_per_w
        pltpu.sync_copy(idx_hbm.at[pl.ds(base, m_per_w)], idx_v)

        def g_fire(j, b):
            pltpu.async_copy(table_hbm.at[idx_v.at[j]], rows_v.at[b], gsems[b])

        def g_wait(j, b):
            pltpu.make_async_copy(
                table_hbm.at[idx_v.at[j]], rows_v.at[b], gsems[b]).wait()

        def o_fire(j, b):
            pltpu.async_copy(
                rows_v.at[b], out_hbm.at[base + j].at[:, pl.ds(0, D)],
                osems[b])

        def o_wait(j, b):
            pltpu.make_async_copy(
                rows_v.at[b], out_hbm.at[base + j].at[:, pl.ds(0, D)],
                osems[b]).wait()

        for b in range(_LA):
            g_fire(b, b)

        def step(j, b, first, last):
            b2 = (b + _LA) % _NB
            g_wait(j, b)
            o_fire(j, b)
            if not first:
                o_wait(j - _LA, b2)
            if not last:
                g_fire(j + _LA, b2)

        for b in range(_NB):
            step(b, b, first=(b < _LA), last=False)

        def group(g, carry):
            j0 = g * _NB
            for b in range(_NB):
                step(j0 + b, b, first=False, last=False)
            return carry

        lax.fori_loop(1, m_per_w // _NB - 1, group, 0)

        j0 = m_per_w - _NB
        for b in range(_NB):
            step(j0 + b, b, first=False, last=(b >= _NB - _LA))

        for j in range(m_per_w - _LA, m_per_w):
            o_wait(j, j % _NB)

    return emb


def kernel(X, E):
    M, S = X.shape
    V, D = E.shape
    out_pad = _emb_call(M, S, D, 128)(E, X)
    return lax.slice(out_pad, (0, 0, 0), (M, S, D))
